# Initial kernel scaffold; baseline (speedup 1.0000x reference)
import jax
import jax.numpy as jnp
from jax.experimental import pallas as pl


def kernel(fw_adj_info, bw_adj_info, feature_info, batch_nodes, embed_table, W_ih_f, W_hh_f, b_f, W_ih_b, W_hh_b, b_b, fw_agg_W, bw_agg_W):
    raise NotImplementedError("write your pallas kernel here")


# speedup vs baseline: 1.3185x; 1.3185x over previous
import functools

import jax
import jax.numpy as jnp
from jax import lax
from jax.experimental import pallas as pl
from jax.experimental.pallas import tpu as pltpu
from jax.experimental.pallas import tpu_sc as plsc

N_NODES = 10000
ADJ_W = 32
EMB = 128
HID = 128
LAYERS = 3
SENT = 100
TLEN = 100

NC = 2
NS = 16
NW = NC * NS


def _sc_gather_small(V, D, n_chunks, chunk, dtype):
    rows_pw = n_chunks * chunk
    mesh = plsc.VectorSubcoreMesh(core_axis_name="c", subcore_axis_name="s")

    @functools.partial(
        pl.kernel,
        out_type=jax.ShapeDtypeStruct((NW * rows_pw, D), dtype),
        mesh=mesh,
        scratch_types=[
            pltpu.VMEM((n_chunks, chunk), jnp.int32),
            pltpu.VMEM((rows_pw, D), dtype),
            pltpu.SemaphoreType.DMA,
        ],
    )
    def k(table_h, idx_h, out_h, idx_v, rows_v, sem):
        wid = lax.axis_index("s") * NC + lax.axis_index("c")
        base = wid * rows_pw
        pltpu.sync_copy(idx_h.at[wid], idx_v)
        handles = []
        for ci in range(n_chunks):
            handles.append(
                pltpu.async_copy(
                    table_h.at[idx_v.at[ci]],
                    rows_v.at[pl.ds(ci * chunk, chunk)],
                    sem,
                )
            )
        for h in handles:
            h.wait()
        pltpu.sync_copy(rows_v, out_h.at[pl.ds(base, rows_pw)])

    return k


def _sc_gather_big(V, D, n_chunks, chunk, dtype):
    rows_pw = n_chunks * chunk
    mesh = plsc.VectorSubcoreMesh(core_axis_name="c", subcore_axis_name="s")
    assert n_chunks % 2 == 0

    @functools.partial(
        pl.kernel,
        out_type=jax.ShapeDtypeStruct((NW * rows_pw, D), dtype),
        mesh=mesh,
        scratch_types=[
            pltpu.VMEM((n_chunks, chunk), jnp.int32),
            pltpu.VMEM((2, chunk, D), dtype),
            pltpu.SemaphoreType.DMA,
            pltpu.SemaphoreType.DMA,
        ],
    )
    def k(table_h, idx_h, out_h, idx_v, rows_v, gsem, osem):
        wid = lax.axis_index("s") * NC + lax.axis_index("c")
        base = wid * rows_pw
        pltpu.sync_copy(idx_h.at[wid], idx_v)

        def body(g, carry):
            c0 = g * 2
            h0 = pltpu.async_copy(table_h.at[idx_v.at[c0]], rows_v.at[0], gsem)
            h1 = pltpu.async_copy(table_h.at[idx_v.at[c0 + 1]], rows_v.at[1], gsem)
            h0.wait()
            o0 = pltpu.async_copy(
                rows_v.at[0], out_h.at[pl.ds(base + c0 * chunk, chunk)], osem)
            h1.wait()
            o1 = pltpu.async_copy(
                rows_v.at[1], out_h.at[pl.ds(base + (c0 + 1) * chunk, chunk)], osem)
            o0.wait()
            o1.wait()
            return carry

        lax.fori_loop(0, n_chunks // 2, body, 0)

    return k


def _pad_idx(idx_flat, n_chunks, chunk):
    total = NW * n_chunks * chunk
    idx_flat = idx_flat.astype(jnp.int32)
    pad = total - idx_flat.shape[0]
    if pad:
        idx_flat = jnp.concatenate([idx_flat, jnp.zeros((pad,), jnp.int32)])
    return idx_flat.reshape(NW, n_chunks, chunk)


def _gather_rows(table, idx_flat, big=False):
    B = idx_flat.shape[0]
    V, D = table.shape
    if big:
        chunk = 128
        n_chunks = -(-B // (NW * chunk))
        if n_chunks % 2:
            n_chunks += 1
        fn = _sc_gather_big(V, D, n_chunks, chunk, table.dtype)
    else:
        chunk = 64
        n_chunks = -(-B // (NW * chunk))
        fn = _sc_gather_small(V, D, n_chunks, chunk, table.dtype)
    idx3 = _pad_idx(idx_flat, n_chunks, chunk)
    out = fn(table, idx3)
    return out[:B]


def _lstm_body(emb_ref, wif_ref, whf_ref, bf_ref, wib_ref, whb_ref, bb_ref,
               outf_ref, outb_ref):
    H2 = HID // 2

    def run(wi_ref, wh_ref, b_ref, out_ref, reverse):
        def step(s, carry):
            h, c = carry
            t = (TLEN - 1 - s) if reverse else s
            xt = emb_ref[t]
            g = (jnp.dot(xt, wi_ref[...], preferred_element_type=jnp.float32)
                 + jnp.dot(h, wh_ref[...], preferred_element_type=jnp.float32)
                 + b_ref[...])
            i = jax.nn.sigmoid(g[:, 0:H2])
            f = jax.nn.sigmoid(g[:, H2:2 * H2])
            gg = jnp.tanh(g[:, 2 * H2:3 * H2])
            o = jax.nn.sigmoid(g[:, 3 * H2:4 * H2])
            c = f * c + i * gg
            h = o * jnp.tanh(c)
            out_ref[t] = h
            return (h, c)

        z = jnp.zeros((SENT, H2), jnp.float32)
        lax.fori_loop(0, TLEN, step, (z, z))

    run(wif_ref, whf_ref, bf_ref, outf_ref, False)
    run(wib_ref, whb_ref, bb_ref, outb_ref, True)


def _run_lstm(embT, W_ih_f, W_hh_f, b_f, W_ih_b, W_hh_b, b_b):
    H2 = HID // 2
    out_shapes = (
        jax.ShapeDtypeStruct((TLEN, SENT, H2), jnp.float32),
        jax.ShapeDtypeStruct((TLEN, SENT, H2), jnp.float32),
    )
    return pl.pallas_call(
        _lstm_body,
        out_shape=out_shapes,
    )(embT.reshape(TLEN, SENT, EMB), W_ih_f.T, W_hh_f.T, b_f.reshape(1, -1),
      W_ih_b.T, W_hh_b.T, b_b.reshape(1, -1))


_AGG_BLK = 400


def _agg0_body(h_ref, neigh_ref, w_ref, out_ref, len_ref):
    neigh = neigh_ref[...]
    r = jnp.sum(jax.nn.relu(neigh), axis=2)
    lens = jnp.sum(jnp.sign(r), axis=1, keepdims=True)
    len_ref[...] = lens
    s = jnp.sum(neigh, axis=1)
    means = s / jnp.maximum(lens, 1.0)
    acc = (jnp.dot(h_ref[...], w_ref[0:HID, :], preferred_element_type=jnp.float32)
           + jnp.dot(means, w_ref[HID:2 * HID, :], preferred_element_type=jnp.float32))
    out_ref[...] = jax.nn.relu(acc)


def _aggk_body(h_ref, neigh_ref, len_ref, w_ref, out_ref):
    s = jnp.sum(neigh_ref[...], axis=1)
    means = s / jnp.maximum(len_ref[...], 1.0)
    acc = (jnp.dot(h_ref[...], w_ref[0:HID, :], preferred_element_type=jnp.float32)
           + jnp.dot(means, w_ref[HID:2 * HID, :], preferred_element_type=jnp.float32))
    out_ref[...] = jax.nn.relu(acc)


def _agg_layer0(h, neigh3, W):
    nblk = N_NODES // _AGG_BLK
    return pl.pallas_call(
        _agg0_body,
        grid=(nblk,),
        in_specs=[
            pl.BlockSpec((_AGG_BLK, HID), lambda i: (i, 0)),
            pl.BlockSpec((_AGG_BLK, ADJ_W, HID), lambda i: (i, 0, 0)),
            pl.BlockSpec((2 * HID, HID), lambda i: (0, 0)),
        ],
        out_specs=[
            pl.BlockSpec((_AGG_BLK, HID), lambda i: (i, 0)),
            pl.BlockSpec((_AGG_BLK, 1), lambda i: (i, 0)),
        ],
        out_shape=[
            jax.ShapeDtypeStruct((N_NODES, HID), jnp.float32),
            jax.ShapeDtypeStruct((N_NODES, 1), jnp.float32),
        ],
    )(h, neigh3, W)


def _agg_layerk(h, neigh3, lens, W):
    nblk = N_NODES // _AGG_BLK
    return pl.pallas_call(
        _aggk_body,
        grid=(nblk,),
        in_specs=[
            pl.BlockSpec((_AGG_BLK, HID), lambda i: (i, 0)),
            pl.BlockSpec((_AGG_BLK, ADJ_W, HID), lambda i: (i, 0, 0)),
            pl.BlockSpec((_AGG_BLK, 1), lambda i: (i, 0)),
            pl.BlockSpec((2 * HID, HID), lambda i: (0, 0)),
        ],
        out_specs=pl.BlockSpec((_AGG_BLK, HID), lambda i: (i, 0)),
        out_shape=jax.ShapeDtypeStruct((N_NODES, HID), jnp.float32),
    )(h, neigh3, lens, W)


def _pool_body(fw_ref, bw_ref, out_ref):
    pf = jnp.max(fw_ref[...], axis=1)
    pb = jnp.max(bw_ref[...], axis=1)
    out_ref[...] = jnp.concatenate([pf, pb], axis=-1)


def _run_pool(fw3, bw3):
    nb, tb = fw3.shape[0], fw3.shape[1]
    return pl.pallas_call(
        _pool_body,
        out_shape=jax.ShapeDtypeStruct((nb, 2 * HID), jnp.float32),
    )(fw3, bw3)


def _remap(n):
    return (n % SENT) * TLEN + n // SENT


def kernel(fw_adj_info, bw_adj_info, feature_info, batch_nodes, embed_table,
           W_ih_f, W_hh_f, b_f, W_ih_b, W_hh_b, b_b, fw_agg_W, bw_agg_W):
    nodes = batch_nodes.reshape(-1).astype(jnp.int32)

    idxT = feature_info.T.reshape(-1).astype(jnp.int32)
    embT = _gather_rows(embed_table, idxT)

    out_f, out_b = _run_lstm(embT, W_ih_f, W_hh_f, b_f, W_ih_b, W_hh_b, b_b)
    table_pm = jnp.concatenate(
        [out_f.reshape(-1, HID // 2), out_b.reshape(-1, HID // 2)],
        axis=-1)
    output_vector = table_pm.reshape(TLEN, SENT, HID).swapaxes(0, 1)

    h0 = _gather_rows(table_pm, _remap(nodes))
    fw_adj_pad = jnp.pad(fw_adj_info.astype(jnp.int32), ((0, 0), (0, 128 - ADJ_W)))
    bw_adj_pad = jnp.pad(bw_adj_info.astype(jnp.int32), ((0, 0), (0, 128 - ADJ_W)))
    fw_sampled = _gather_rows(fw_adj_pad, nodes)[:, :ADJ_W]
    bw_sampled = _gather_rows(bw_adj_pad, nodes)[:, :ADJ_W]

    def chain(sampled, W3):
        flat = sampled.reshape(-1)
        neigh0 = _gather_rows(table_pm, _remap(flat), big=True)
        h, lens = _agg_layer0(h0, neigh0.reshape(N_NODES, ADJ_W, HID), W3[0])
        for layer in range(1, LAYERS):
            neigh = _gather_rows(h, flat, big=True)
            h = _agg_layerk(h, neigh.reshape(N_NODES, ADJ_W, HID), lens, W3[layer])
        return h

    fw_hidden = chain(fw_sampled, fw_agg_W)
    bw_hidden = chain(bw_sampled, bw_agg_W)

    nb_rows, nb_cols = batch_nodes.shape
    fw3 = fw_hidden.reshape(nb_rows, nb_cols, HID)
    bw3 = bw_hidden.reshape(nb_rows, nb_cols, HID)
    pooled = _run_pool(fw3, bw3)
    hidden = jnp.concatenate([fw3, bw3], axis=2)
    graph_embedding = pooled.reshape(-1, HID)
    return hidden, graph_embedding, output_vector
